# baseline (speedup 1.0000x reference)
import jax
import jax.numpy as jnp
from jax.experimental import pallas as pl
from jax.experimental.pallas import tpu as pltpu

_TMP = 0.1
_FMP = 0.1


def _mask_body(t0_ref, f0_ref, x_ref, o_ref, *, tlen, flen):
    b = pl.program_id(0)
    t0 = t0_ref[b]
    f0 = f0_ref[b]
    x = x_ref[0, 0]
    col = jax.lax.broadcasted_iota(jnp.int32, x.shape, 1)
    row = jax.lax.broadcasted_iota(jnp.int32, x.shape, 0)
    tmask = (col >= t0) & (col < t0 + tlen)
    fmask = (row >= f0) & (row < f0 + flen)
    o_ref[0, 0] = jnp.where(tmask | fmask, jnp.float32(0.0), x)


def kernel(spec):
    B, C, Fd, T = spec.shape
    tlen = int(T * _TMP)
    flen = int(Fd * _FMP)
    t0 = jax.random.randint(
        jax.random.fold_in(jax.random.key(1), 0), (B,), 0, max(1, T - tlen + 1)
    ).astype(jnp.int32)
    f0 = jax.random.randint(
        jax.random.fold_in(jax.random.key(1), 1), (B,), 0, max(1, Fd - flen + 1)
    ).astype(jnp.int32)

    import functools

    body = functools.partial(_mask_body, tlen=tlen, flen=flen)
    return pl.pallas_call(
        body,
        grid=(B,),
        in_specs=[
            pl.BlockSpec(memory_space=pltpu.SMEM),
            pl.BlockSpec(memory_space=pltpu.SMEM),
            pl.BlockSpec((1, C, Fd, T), lambda b: (b, 0, 0, 0)),
        ],
        out_specs=pl.BlockSpec((1, C, Fd, T), lambda b: (b, 0, 0, 0)),
        out_shape=jax.ShapeDtypeStruct(spec.shape, spec.dtype),
    )(t0, f0, spec)
